# baseline (speedup 1.0000x reference)
import dataclasses
import functools

import jax
import jax.numpy as jnp
from jax.experimental import pallas as pl
from jax.experimental.pallas import tpu as pltpu
from jax.experimental.pallas import tpu_sc as plsc

B, D, R = 512, 512, 256
TOP_K_FACTS, TOP_K_RULES, FACT_TEMP = 2, 8, 0.7
NEG = -3.0e38


def _sub_prod(t):
    n = t.shape[0]
    while n > 1:
        h = n // 2
        t = t[:h, :] * t[h:n, :]
        n = h
    return t


def _dot_t(a, b):
    return jax.lax.dot_general(
        a, b, (((1,), (1,)), ((), ())),
        precision=jax.lax.Precision.HIGHEST,
        preferred_element_type=jnp.float32)


def _acts_body(facts_ref, fl_ref, agg_ref, rs_ref, w_ref,
               act_ref, proj_ref, mask_ref, andt_ref, ort_ref):
    f = facts_ref[...]
    fl = fl_ref[...]

    z = fl * (1.0 / FACT_TEMP)
    z = z - jnp.max(z, axis=1, keepdims=True)
    e = jnp.exp(z)
    p = e / jnp.sum(e, axis=1, keepdims=True)
    mask = jnp.minimum(TOP_K_FACTS * p, 1.0)
    mask_ref[...] = mask
    denom = jnp.sum(mask, axis=1, keepdims=True) + 1e-8

    kofnt = _dot_t(mask, f) / denom

    ft = f.T
    ft_hi, ft_lo = ft[: D // 2], ft[D // 2:]

    def rule(r, carry):
        m_col = mask_ref[pl.ds(r, 1), :].T
        m_hi, m_lo = m_col[: D // 2], m_col[D // 2:]
        sh = ft_hi * m_hi
        sl = ft_lo * m_lo
        a1 = (sh + (1.0 - m_hi)) * (sl + (1.0 - m_lo))
        o1 = ((1.0 - sh) + 1e-8) * ((1.0 - sl) + 1e-8)
        andt_ref[pl.ds(r, 1), :] = _sub_prod(a1)
        ort_ref[pl.ds(r, 1), :] = _sub_prod(o1)
        return carry

    jax.lax.fori_loop(0, R, rule, 0, unroll=8)

    aw = agg_ref[...]
    aw = aw - jnp.max(aw, axis=1, keepdims=True)
    ea = jnp.exp(aw)
    w = ea / jnp.sum(ea, axis=1, keepdims=True)
    strength = jax.nn.sigmoid(rs_ref[...])

    mixedt = (andt_ref[...] * w[:, 0:1]
              + (1.0 - ort_ref[...]) * w[:, 1:2]
              + kofnt * w[:, 2:3]) * strength
    act_ref[...] = mixedt.T
    proj_ref[...] = _dot_t(f, w_ref[...])


def _sc_gate(act):
    mesh = plsc.VectorSubcoreMesh(core_axis_name="core",
                                  subcore_axis_name="subcore")
    n_units = 2 * 16
    rows_per_unit = B // n_units
    n_chunks = R // 16

    cp = pltpu.CompilerParams()
    if "needs_layout_passes" in pltpu.CompilerParams.__dataclass_fields__:
        cp = dataclasses.replace(cp, needs_layout_passes=False)

    @pl.kernel(out_type=jax.ShapeDtypeStruct((B, R), jnp.float32),
               mesh=mesh,
               compiler_params=cp,
               scratch_types=[pltpu.VMEM((R,), jnp.float32),
                              pltpu.VMEM((R,), jnp.float32),
                              pltpu.VMEM((R,), jnp.float32)])
    def gate_kernel(act_hbm, o_hbm, orig_ref, work_ref, out_ref):
        unit = jax.lax.axis_index("core") * 16 + jax.lax.axis_index("subcore")

        @pl.loop(0, rows_per_unit)
        def _(i):
            r = unit * rows_per_unit + i
            pltpu.sync_copy(act_hbm.at[r], orig_ref)
            for c in range(n_chunks):
                work_ref[pl.ds(16 * c, 16)] = orig_ref[pl.ds(16 * c, 16)]

            remaining = jnp.int32(TOP_K_RULES)
            thresh = jnp.float32(3.0e38)
            for _it in range(TOP_K_RULES):
                vs = [work_ref[pl.ds(16 * c, 16)] for c in range(n_chunks)]
                mv = functools.reduce(jnp.maximum, vs)
                m = jnp.max(mv)
                cnt = jnp.int32(0)
                for c in range(n_chunks):
                    cnt = cnt + jnp.sum((vs[c] == m).astype(jnp.int32))
                upd = remaining > 0
                thresh = jnp.where(upd, m, thresh)
                remaining = jnp.where(upd, remaining - cnt, remaining)
                for c in range(n_chunks):
                    work_ref[pl.ds(16 * c, 16)] = jnp.where(
                        upd & (vs[c] == m), NEG, vs[c])

            vos = [orig_ref[pl.ds(16 * c, 16)] for c in range(n_chunks)]
            count_gt = jnp.int32(0)
            for c in range(n_chunks):
                count_gt = count_gt + jnp.sum(
                    (vos[c] > thresh).astype(jnp.int32))
            need = jnp.int32(TOP_K_RULES) - count_gt
            carry = jnp.int32(0)
            for c in range(n_chunks):
                eq = vos[c] == thresh
                cums = jax.lax.cumsum(eq.astype(jnp.int32)) + carry
                g = (vos[c] > thresh) | (eq & (cums <= need))
                out_ref[pl.ds(16 * c, 16)] = jnp.where(g, vos[c], 0.0)
                carry = carry + jnp.sum(eq.astype(jnp.int32))
            pltpu.sync_copy(out_ref, o_hbm.at[r])

    return gate_kernel(act)


def _finish_body(proj_ref, gated_ref, gamma_ref, beta_ref, out_ref):
    pre = proj_ref[...] + gated_ref[...]
    mu = jnp.mean(pre, axis=1, keepdims=True)
    cen = pre - mu
    var = jnp.mean(cen * cen, axis=1, keepdims=True)
    out_ref[...] = cen * jax.lax.rsqrt(var + 1e-5) * gamma_ref[...] \
        + beta_ref[...]


@jax.jit
def kernel(facts, fact_logits, aggregator_logits, rule_strength_raw, W_proj,
           ln_gamma, ln_beta):
    rs = rule_strength_raw.reshape(R, 1)
    gamma = ln_gamma.reshape(1, R)
    beta = ln_beta.reshape(1, R)
    act, proj = pl.pallas_call(
        _acts_body,
        out_shape=[jax.ShapeDtypeStruct((B, R), jnp.float32),
                   jax.ShapeDtypeStruct((B, R), jnp.float32)],
        scratch_shapes=[
            pltpu.VMEM((R, D), jnp.float32),
            pltpu.VMEM((R, B), jnp.float32),
            pltpu.VMEM((R, B), jnp.float32),
        ],
    )(facts, fact_logits, aggregator_logits, rs, W_proj)
    gated = _sc_gate(act)
    return pl.pallas_call(
        _finish_body,
        out_shape=jax.ShapeDtypeStruct((B, R), jnp.float32),
    )(proj, gated, gamma, beta)


# speedup vs baseline: 1.0686x; 1.0686x over previous
import dataclasses
import functools

import jax
import jax.numpy as jnp
from jax.experimental import pallas as pl
from jax.experimental.pallas import tpu as pltpu
from jax.experimental.pallas import tpu_sc as plsc

B, D, R = 512, 512, 256
TOP_K_FACTS, TOP_K_RULES, FACT_TEMP = 2, 8, 0.7
NEG = -3.0e38


def _sub_prod(t):
    n = t.shape[0]
    while n > 1:
        h = n // 2
        t = t[:h, :] * t[h:n, :]
        n = h
    return t


def _dot_t(a, b):
    return jax.lax.dot_general(
        a, b, (((1,), (1,)), ((), ())),
        precision=jax.lax.Precision.HIGHEST,
        preferred_element_type=jnp.float32)


def _acts_body(facts_ref, fl_ref, agg_ref, rs_ref, w_ref,
               act_ref, proj_ref, mask_ref, andt_ref, ort_ref):
    f = facts_ref[...]
    fl = fl_ref[...]

    z = fl * (1.0 / FACT_TEMP)
    z = z - jnp.max(z, axis=1, keepdims=True)
    e = jnp.exp(z)
    p = e / jnp.sum(e, axis=1, keepdims=True)
    mask = jnp.minimum(TOP_K_FACTS * p, 1.0)
    mask_ref[...] = mask
    denom = jnp.sum(mask, axis=1, keepdims=True) + 1e-8

    kofnt = _dot_t(mask, f) / denom

    ft = f.T
    ft_hi, ft_lo = ft[: D // 2], ft[D // 2:]

    def rule(r, carry):
        m_col = mask_ref[pl.ds(r, 1), :].T
        m_hi, m_lo = m_col[: D // 2], m_col[D // 2:]
        sh = ft_hi * m_hi
        sl = ft_lo * m_lo
        a1 = (sh + (1.0 - m_hi)) * (sl + (1.0 - m_lo))
        o1 = ((1.0 - sh) + 1e-8) * ((1.0 - sl) + 1e-8)
        andt_ref[pl.ds(r, 1), :] = _sub_prod(a1)
        ort_ref[pl.ds(r, 1), :] = _sub_prod(o1)
        return carry

    jax.lax.fori_loop(0, R, rule, 0, unroll=8)

    aw = agg_ref[...]
    aw = aw - jnp.max(aw, axis=1, keepdims=True)
    ea = jnp.exp(aw)
    w = ea / jnp.sum(ea, axis=1, keepdims=True)
    strength = jax.nn.sigmoid(rs_ref[...])

    mixedt = (andt_ref[...] * w[:, 0:1]
              + (1.0 - ort_ref[...]) * w[:, 1:2]
              + kofnt * w[:, 2:3]) * strength
    act_ref[...] = mixedt.T
    proj_ref[...] = _dot_t(f, w_ref[...])


def _sc_gate(act):
    mesh = plsc.VectorSubcoreMesh(core_axis_name="core",
                                  subcore_axis_name="subcore")
    n_units = 2 * 16
    rows_per_unit = B // n_units
    n_chunks = R // 16

    cp = pltpu.CompilerParams()
    if "needs_layout_passes" in pltpu.CompilerParams.__dataclass_fields__:
        cp = dataclasses.replace(cp, needs_layout_passes=False)

    @pl.kernel(out_type=jax.ShapeDtypeStruct((B, R), jnp.float32),
               mesh=mesh,
               compiler_params=cp,
               scratch_types=[pltpu.VMEM((rows_per_unit, R), jnp.float32),
                              pltpu.VMEM((rows_per_unit, R), jnp.float32),
                              pltpu.VMEM((R,), jnp.float32)])
    def gate_kernel(act_hbm, o_hbm, orig_ref, out_ref, work_ref):
        unit = jax.lax.axis_index("core") * 16 + jax.lax.axis_index("subcore")
        r0 = unit * rows_per_unit
        pltpu.sync_copy(act_hbm.at[pl.ds(r0, rows_per_unit)], orig_ref)

        @pl.loop(0, rows_per_unit)
        def _(i):
            for c in range(n_chunks):
                work_ref[pl.ds(16 * c, 16)] = orig_ref[i, pl.ds(16 * c, 16)]

            remaining = jnp.int32(TOP_K_RULES)
            thresh = jnp.float32(3.0e38)
            for _it in range(TOP_K_RULES):
                vs = [work_ref[pl.ds(16 * c, 16)] for c in range(n_chunks)]
                mv = functools.reduce(jnp.maximum, vs)
                m = jnp.max(mv)
                cnt = jnp.int32(0)
                for c in range(n_chunks):
                    cnt = cnt + jnp.sum((vs[c] == m).astype(jnp.int32))
                upd = remaining > 0
                thresh = jnp.where(upd, m, thresh)
                remaining = jnp.where(upd, remaining - cnt, remaining)
                for c in range(n_chunks):
                    work_ref[pl.ds(16 * c, 16)] = jnp.where(
                        upd & (vs[c] == m), NEG, vs[c])

            vos = [orig_ref[i, pl.ds(16 * c, 16)] for c in range(n_chunks)]
            count_gt = jnp.int32(0)
            for c in range(n_chunks):
                count_gt = count_gt + jnp.sum(
                    (vos[c] > thresh).astype(jnp.int32))
            need = jnp.int32(TOP_K_RULES) - count_gt
            carry = jnp.int32(0)
            for c in range(n_chunks):
                eq = vos[c] == thresh
                cums = jax.lax.cumsum(eq.astype(jnp.int32)) + carry
                g = (vos[c] > thresh) | (eq & (cums <= need))
                out_ref[i, pl.ds(16 * c, 16)] = jnp.where(g, vos[c], 0.0)
                carry = carry + jnp.sum(eq.astype(jnp.int32))

        pltpu.sync_copy(out_ref, o_hbm.at[pl.ds(r0, rows_per_unit)])

    return gate_kernel(act)


def _finish_body(proj_ref, gated_ref, gamma_ref, beta_ref, out_ref):
    pre = proj_ref[...] + gated_ref[...]
    mu = jnp.mean(pre, axis=1, keepdims=True)
    cen = pre - mu
    var = jnp.mean(cen * cen, axis=1, keepdims=True)
    out_ref[...] = cen * jax.lax.rsqrt(var + 1e-5) * gamma_ref[...] \
        + beta_ref[...]


@jax.jit
def kernel(facts, fact_logits, aggregator_logits, rule_strength_raw, W_proj,
           ln_gamma, ln_beta):
    rs = rule_strength_raw.reshape(R, 1)
    gamma = ln_gamma.reshape(1, R)
    beta = ln_beta.reshape(1, R)
    act, proj = pl.pallas_call(
        _acts_body,
        out_shape=[jax.ShapeDtypeStruct((B, R), jnp.float32),
                   jax.ShapeDtypeStruct((B, R), jnp.float32)],
        scratch_shapes=[
            pltpu.VMEM((R, D), jnp.float32),
            pltpu.VMEM((R, B), jnp.float32),
            pltpu.VMEM((R, B), jnp.float32),
        ],
    )(facts, fact_logits, aggregator_logits, rs, W_proj)
    gated = _sc_gate(act)
    return pl.pallas_call(
        _finish_body,
        out_shape=jax.ShapeDtypeStruct((B, R), jnp.float32),
    )(proj, gated, gamma, beta)


# speedup vs baseline: 1.0757x; 1.0067x over previous
import dataclasses
import functools

import jax
import jax.numpy as jnp
from jax.experimental import pallas as pl
from jax.experimental.pallas import tpu as pltpu
from jax.experimental.pallas import tpu_sc as plsc

B, D, R = 512, 512, 256
TOP_K_FACTS, TOP_K_RULES, FACT_TEMP = 2, 8, 0.7
NEG = -3.0e38


def _sub_prod(t):
    n = t.shape[0]
    while n > 1:
        h = n // 2
        t = t[:h, :] * t[h:n, :]
        n = h
    return t


def _dot_t(a, b):
    return jax.lax.dot_general(
        a, b, (((1,), (1,)), ((), ())),
        precision=jax.lax.Precision.HIGHEST,
        preferred_element_type=jnp.float32)


def _acts_body(facts_ref, fl_ref, agg_ref, rs_ref, w_ref,
               act_ref, proj_ref, mask_ref, andt_ref, ort_ref):
    f = facts_ref[...]
    fl = fl_ref[...]

    z = fl * (1.0 / FACT_TEMP)
    z = z - jnp.max(z, axis=1, keepdims=True)
    e = jnp.exp(z)
    p = e / jnp.sum(e, axis=1, keepdims=True)
    mask = jnp.minimum(TOP_K_FACTS * p, 1.0)
    mask_ref[...] = mask
    denom = jnp.sum(mask, axis=1, keepdims=True) + 1e-8

    kofnt = _dot_t(mask, f) / denom

    ft = f.T
    ft_hi, ft_lo = ft[: D // 2], ft[D // 2:]

    def rule(r, carry):
        m_col = mask_ref[pl.ds(r, 1), :].T
        m_hi, m_lo = m_col[: D // 2], m_col[D // 2:]
        sh = ft_hi * m_hi
        sl = ft_lo * m_lo
        a1 = (sh + (1.0 - m_hi)) * (sl + (1.0 - m_lo))
        o1 = ((1.0 - sh) + 1e-8) * ((1.0 - sl) + 1e-8)
        andt_ref[pl.ds(r, 1), :] = _sub_prod(a1)
        ort_ref[pl.ds(r, 1), :] = _sub_prod(o1)
        return carry

    jax.lax.fori_loop(0, R, rule, 0, unroll=8)

    aw = agg_ref[...]
    aw = aw - jnp.max(aw, axis=1, keepdims=True)
    ea = jnp.exp(aw)
    w = ea / jnp.sum(ea, axis=1, keepdims=True)
    strength = jax.nn.sigmoid(rs_ref[...])

    mixedt = (andt_ref[...] * w[:, 0:1]
              + (1.0 - ort_ref[...]) * w[:, 1:2]
              + kofnt * w[:, 2:3]) * strength
    act_ref[...] = mixedt.T
    proj_ref[...] = _dot_t(f, w_ref[...])


def _sc_gate(act):
    mesh = plsc.VectorSubcoreMesh(core_axis_name="core",
                                  subcore_axis_name="subcore")
    n_units = 2 * 16
    rows_per_unit = B // n_units
    n_chunks = R // 16

    cp = pltpu.CompilerParams()
    if "needs_layout_passes" in pltpu.CompilerParams.__dataclass_fields__:
        cp = dataclasses.replace(cp, needs_layout_passes=False)

    @pl.kernel(out_type=jax.ShapeDtypeStruct((B, R), jnp.float32),
               mesh=mesh,
               compiler_params=cp,
               scratch_types=[pltpu.VMEM((rows_per_unit, R), jnp.float32),
                              pltpu.VMEM((rows_per_unit, R), jnp.float32)])
    def gate_kernel(act_hbm, o_hbm, orig_ref, out_ref):
        unit = jax.lax.axis_index("core") * 16 + jax.lax.axis_index("subcore")
        r0 = unit * rows_per_unit
        pltpu.sync_copy(act_hbm.at[pl.ds(r0, rows_per_unit)], orig_ref)

        @pl.loop(0, rows_per_unit)
        def _(i):
            vs = [orig_ref[i, pl.ds(16 * c, 16)] for c in range(n_chunks)]
            remaining = jnp.int32(TOP_K_RULES)
            thresh = jnp.float32(3.0e38)
            for _it in range(TOP_K_RULES):
                mv = functools.reduce(jnp.maximum, vs)
                m = jnp.max(mv)
                cnt = jnp.int32(0)
                for c in range(n_chunks):
                    cnt = cnt + jnp.sum((vs[c] == m).astype(jnp.int32))
                upd = remaining > 0
                thresh = jnp.where(upd, m, thresh)
                remaining = jnp.where(upd, remaining - cnt, remaining)
                for c in range(n_chunks):
                    vs[c] = jnp.where(upd & (vs[c] == m), NEG, vs[c])

            vos = [orig_ref[i, pl.ds(16 * c, 16)] for c in range(n_chunks)]
            count_gt = jnp.int32(0)
            for c in range(n_chunks):
                count_gt = count_gt + jnp.sum(
                    (vos[c] > thresh).astype(jnp.int32))
            need = jnp.int32(TOP_K_RULES) - count_gt
            carry = jnp.int32(0)
            for c in range(n_chunks):
                eq = vos[c] == thresh
                cums = jax.lax.cumsum(eq.astype(jnp.int32)) + carry
                g = (vos[c] > thresh) | (eq & (cums <= need))
                out_ref[i, pl.ds(16 * c, 16)] = jnp.where(g, vos[c], 0.0)
                carry = carry + jnp.sum(eq.astype(jnp.int32))

        pltpu.sync_copy(out_ref, o_hbm.at[pl.ds(r0, rows_per_unit)])

    return gate_kernel(act)


def _finish_body(proj_ref, gated_ref, gamma_ref, beta_ref, out_ref):
    pre = proj_ref[...] + gated_ref[...]
    mu = jnp.mean(pre, axis=1, keepdims=True)
    cen = pre - mu
    var = jnp.mean(cen * cen, axis=1, keepdims=True)
    out_ref[...] = cen * jax.lax.rsqrt(var + 1e-5) * gamma_ref[...] \
        + beta_ref[...]


@jax.jit
def kernel(facts, fact_logits, aggregator_logits, rule_strength_raw, W_proj,
           ln_gamma, ln_beta):
    rs = rule_strength_raw.reshape(R, 1)
    gamma = ln_gamma.reshape(1, R)
    beta = ln_beta.reshape(1, R)
    act, proj = pl.pallas_call(
        _acts_body,
        out_shape=[jax.ShapeDtypeStruct((B, R), jnp.float32),
                   jax.ShapeDtypeStruct((B, R), jnp.float32)],
        scratch_shapes=[
            pltpu.VMEM((R, D), jnp.float32),
            pltpu.VMEM((R, B), jnp.float32),
            pltpu.VMEM((R, B), jnp.float32),
        ],
    )(facts, fact_logits, aggregator_logits, rs, W_proj)
    gated = _sc_gate(act)
    return pl.pallas_call(
        _finish_body,
        out_shape=jax.ShapeDtypeStruct((B, R), jnp.float32),
    )(proj, gated, gamma, beta)
